# Initial kernel scaffold; baseline (speedup 1.0000x reference)
import functools

import jax
import jax.numpy as jnp
from jax import lax
from jax.experimental import pallas as pl
from jax.experimental.pallas import tpu as pltpu
from jax.experimental.pallas import tpu_sc as plsc

_B = 4
_T = 2048
_D = 128
_NB = _B * _T
_NW = 32
_BPW = _NB // _NW

_mesh = plsc.VectorSubcoreMesh(core_axis_name="c", subcore_axis_name="s")


@functools.partial(
    pl.kernel,
    mesh=_mesh,
    out_type=jax.ShapeDtypeStruct((_NB, _D), jnp.float32),
    scratch_types=[
        pltpu.VMEM((_BPW,), jnp.int32),
        pltpu.VMEM((_BPW,), jnp.int32),
        pltpu.VMEM((_BPW, _D), jnp.float32),
        pltpu.VMEM((_BPW, _D), jnp.float32),
        pltpu.SemaphoreType.DMA,
    ],
)
def _embed(x_hbm, tok_hbm, pos_hbm, ident_hbm, out_hbm,
           idx_v, ident_v, acc_v, pos_v, sem):
    wid = lax.axis_index("s") * 2 + lax.axis_index("c")
    base = wid * _BPW
    pbase = lax.rem(base, _T)
    pltpu.sync_copy(x_hbm.at[pl.ds(base, _BPW)], idx_v)
    pltpu.sync_copy(ident_hbm, ident_v)
    pltpu.async_copy(tok_hbm.at[idx_v], acc_v, sem).wait()
    pltpu.sync_copy(pos_hbm.at[pl.ds(pbase, _BPW)], pos_v)
    pltpu.sync_copy(pos_v, acc_v.at[ident_v], add=True)
    pltpu.sync_copy(acc_v, out_hbm.at[pl.ds(base, _BPW)])


def kernel(x, tok_table, pos_table):
    x_flat = x.reshape(-1).astype(jnp.int32)
    ident = jnp.arange(_BPW, dtype=jnp.int32)
    out = _embed(x_flat, tok_table, pos_table, ident)
    return out.reshape(_B, _T, _D)


# speedup vs baseline: 1.2372x; 1.2372x over previous
import functools

import jax
import jax.numpy as jnp
from jax import lax
from jax.experimental import pallas as pl
from jax.experimental.pallas import tpu as pltpu
from jax.experimental.pallas import tpu_sc as plsc

_B = 4
_T = 2048
_D = 128
_NB = _B * _T
_NW = 32
_BPW = _NB // _NW

_mesh = plsc.VectorSubcoreMesh(core_axis_name="c", subcore_axis_name="s")


@functools.partial(
    pl.kernel,
    mesh=_mesh,
    out_type=jax.ShapeDtypeStruct((_NB, _D), jnp.float32),
    scratch_types=[
        pltpu.VMEM((_BPW,), jnp.int32),
        pltpu.VMEM((_BPW, _D), jnp.float32),
        pltpu.VMEM((_BPW, _D), jnp.float32),
        pltpu.SemaphoreType.DMA,
    ],
)
def _embed(x_hbm, tok_hbm, pos_hbm, out_hbm,
           idx_v, acc_v, pos_v, sem):
    wid = lax.axis_index("s") * 2 + lax.axis_index("c")
    base = wid * _BPW
    pbase = lax.rem(base, _T)
    pltpu.sync_copy(x_hbm.at[pl.ds(base, _BPW)], idx_v)
    pltpu.sync_copy(pos_hbm.at[pl.ds(pbase, _BPW)], pos_v)
    pltpu.async_copy(tok_hbm.at[idx_v], acc_v, sem).wait()

    @pl.loop(0, _BPW)
    def _(r):
        @pl.loop(0, _D, step=16)
        def _(c):
            slc = (pl.ds(r, 1), pl.ds(c, 16))
            acc_v.at[*slc][...] = acc_v.at[*slc][...] + pos_v.at[*slc][...]

    pltpu.sync_copy(acc_v, out_hbm.at[pl.ds(base, _BPW)])


def kernel(x, tok_table, pos_table):
    x_flat = x.reshape(-1).astype(jnp.int32)
    out = _embed(x_flat, tok_table, pos_table)
    return out.reshape(_B, _T, _D)


# speedup vs baseline: 1.3441x; 1.0864x over previous
import functools

import jax
import jax.numpy as jnp
from jax import lax
from jax.experimental import pallas as pl
from jax.experimental.pallas import tpu as pltpu
from jax.experimental.pallas import tpu_sc as plsc

_B = 4
_T = 2048
_D = 128
_NB = _B * _T
_NW = 32
_BPW = _NB // _NW
_CH = 64
_NCH = _BPW // _CH

_mesh = plsc.VectorSubcoreMesh(core_axis_name="c", subcore_axis_name="s")


@functools.partial(
    pl.kernel,
    mesh=_mesh,
    out_type=jax.ShapeDtypeStruct((_NB, _D), jnp.float32),
    scratch_types=(
        [pltpu.VMEM((_BPW,), jnp.int32)]
        + [pltpu.VMEM((_CH, _D), jnp.float32) for _ in range(_NCH)]
        + [pltpu.SemaphoreType.DMA for _ in range(2 * _NCH)]
        + [pltpu.SemaphoreType.DMA]
    ),
)
def _embed(x_hbm, tok_hbm, pos_hbm, out_hbm, idx_v, *bufs_and_sems):
    accs = bufs_and_sems[:_NCH]
    psems = bufs_and_sems[_NCH:2 * _NCH]
    gsems = bufs_and_sems[2 * _NCH:3 * _NCH]
    osem = bufs_and_sems[3 * _NCH]

    wid = lax.axis_index("s") * 2 + lax.axis_index("c")
    base = wid * _BPW
    pbase = lax.rem(base, _T)
    pltpu.sync_copy(x_hbm.at[pl.ds(base, _BPW)], idx_v)

    pcps = [
        pltpu.async_copy(pos_hbm.at[pl.ds(pbase + c * _CH, _CH)],
                         accs[c], psems[c])
        for c in range(_NCH)
    ]
    gcps = []
    for c in range(_NCH):
        pcps[c].wait()
        gcps.append(pltpu.async_copy(
            tok_hbm.at[idx_v.at[pl.ds(c * _CH, _CH)]],
            accs[c], gsems[c], add=True))

    ocps = []
    for c in range(_NCH):
        gcps[c].wait()
        ocps.append(pltpu.async_copy(
            accs[c], out_hbm.at[pl.ds(base + c * _CH, _CH)], osem))

    for c in range(_NCH):
        ocps[c].wait()


def kernel(x, tok_table, pos_table):
    x_flat = x.reshape(-1).astype(jnp.int32)
    out = _embed(x_flat, tok_table, pos_table)
    return out.reshape(_B, _T, _D)


# speedup vs baseline: 1.3728x; 1.0214x over previous
import functools

import jax
import jax.numpy as jnp
from jax import lax
from jax.experimental import pallas as pl
from jax.experimental.pallas import tpu as pltpu
from jax.experimental.pallas import tpu_sc as plsc

_B = 4
_T = 2048
_D = 128
_NB = _B * _T
_NW = 32
_BPW = _NB // _NW
_CH = 32
_NCH = _BPW // _CH

_mesh = plsc.VectorSubcoreMesh(core_axis_name="c", subcore_axis_name="s")


@functools.partial(
    pl.kernel,
    mesh=_mesh,
    out_type=jax.ShapeDtypeStruct((_NB, _D), jnp.float32),
    scratch_types=(
        [pltpu.VMEM((_BPW,), jnp.int32)]
        + [pltpu.VMEM((_CH, _D), jnp.float32) for _ in range(_NCH)]
        + [pltpu.SemaphoreType.DMA for _ in range(2 * _NCH)]
        + [pltpu.SemaphoreType.DMA, pltpu.SemaphoreType.DMA]
    ),
)
def _embed(x_hbm, tok_hbm, pos_hbm, out_hbm, idx_v, *bufs_and_sems):
    accs = bufs_and_sems[:_NCH]
    psems = bufs_and_sems[_NCH:2 * _NCH]
    gsems = bufs_and_sems[2 * _NCH:3 * _NCH]
    osem = bufs_and_sems[3 * _NCH]
    isem = bufs_and_sems[3 * _NCH + 1]

    wid = lax.axis_index("s") * 2 + lax.axis_index("c")
    base = wid * _BPW
    pbase = lax.rem(base, _T)

    icp = pltpu.async_copy(x_hbm.at[pl.ds(base, _BPW)], idx_v, isem)
    pcps = [
        pltpu.async_copy(pos_hbm.at[pl.ds(pbase + c * _CH, _CH)],
                         accs[c], psems[c])
        for c in range(_NCH)
    ]
    icp.wait()
    gcps = []
    for c in range(_NCH):
        pcps[c].wait()
        gcps.append(pltpu.async_copy(
            tok_hbm.at[idx_v.at[pl.ds(c * _CH, _CH)]],
            accs[c], gsems[c], add=True))

    ocps = []
    for c in range(_NCH):
        gcps[c].wait()
        ocps.append(pltpu.async_copy(
            accs[c], out_hbm.at[pl.ds(base + c * _CH, _CH)], osem))

    for c in range(_NCH):
        ocps[c].wait()


def kernel(x, tok_table, pos_table):
    x_flat = x.reshape(-1).astype(jnp.int32)
    out = _embed(x_flat, tok_table, pos_table)
    return out.reshape(_B, _T, _D)
